# baseline (speedup 1.0000x reference)
import jax
import jax.numpy as jnp
from jax.experimental import pallas as pl
from jax.experimental.pallas import tpu as pltpu

OUT_D = 128
N = 4096
B = 32


def _body(roi_ref, out_ref):
    bbox = roi_ref[...].astype(jnp.int32)
    x = bbox[:, 1]
    y = bbox[:, 2]
    w = bbox[:, 3]
    h = bbox[:, 4]
    r = jax.lax.broadcasted_iota(jnp.int32, (B, OUT_D, 1), 1)
    c = jax.lax.broadcasted_iota(jnp.int32, (B, 1, OUT_D), 2)
    rind = ((r >= y[:, None, None]) & (r <= (y + h)[:, None, None])).astype(
        jnp.float32
    )
    cind = ((c >= x[:, None, None]) & (c <= (x + w)[:, None, None])).astype(
        jnp.float32
    )
    out_ref[...] = rind * cind


def kernel(output_roi):
    return pl.pallas_call(
        _body,
        grid=(N // B,),
        in_specs=[pl.BlockSpec((B, 6), lambda i: (i, 0))],
        out_specs=pl.BlockSpec((B, OUT_D, OUT_D), lambda i: (i, 0, 0)),
        out_shape=jax.ShapeDtypeStruct((N, OUT_D, OUT_D), jnp.float32),
    )(output_roi)


# speedup vs baseline: 1.0224x; 1.0224x over previous
import jax
import jax.numpy as jnp
from jax.experimental import pallas as pl
from jax.experimental.pallas import tpu as pltpu

OUT_D = 128
N = 4096
B = 32


def _body(roi_ref, out_ref):
    r2 = jax.lax.broadcasted_iota(jnp.uint32, (OUT_D, OUT_D), 0)
    c2 = jax.lax.broadcasted_iota(jnp.uint32, (OUT_D, OUT_D), 1)
    for b in range(B):
        x = roi_ref[b, 0].astype(jnp.uint32)
        y = roi_ref[b, 1].astype(jnp.uint32)
        w = roi_ref[b, 2].astype(jnp.uint32)
        h = roi_ref[b, 3].astype(jnp.uint32)
        inside = ((r2 - y) <= h) & ((c2 - x) <= w)
        out_ref[b] = jnp.where(inside, 1.0, 0.0).astype(jnp.float32)


def kernel(output_roi):
    bbox = output_roi[:, 1:5].astype(jnp.int32)
    return pl.pallas_call(
        _body,
        grid=(N // B,),
        in_specs=[
            pl.BlockSpec((B, 4), lambda i: (i, 0), memory_space=pltpu.SMEM)
        ],
        out_specs=pl.BlockSpec((B, OUT_D, OUT_D), lambda i: (i, 0, 0)),
        out_shape=jax.ShapeDtypeStruct((N, OUT_D, OUT_D), jnp.float32),
    )(bbox)


# speedup vs baseline: 1.3695x; 1.3395x over previous
import jax
import jax.numpy as jnp
from jax.experimental import pallas as pl
from jax.experimental.pallas import tpu as pltpu

OUT_D = 128
N = 4096
B = 64


def _body(roi_ref, out_ref):
    r2 = jax.lax.broadcasted_iota(jnp.uint32, (OUT_D, OUT_D), 0)
    c2 = jax.lax.broadcasted_iota(jnp.uint32, (OUT_D, OUT_D), 1)
    for b in range(B):
        x = roi_ref[b, 0].astype(jnp.uint32)
        y = roi_ref[b, 1].astype(jnp.uint32)
        w = roi_ref[b, 2].astype(jnp.uint32)
        h = roi_ref[b, 3].astype(jnp.uint32)
        inside = ((r2 - y) <= h) & ((c2 - x) <= w)
        out_ref[b] = jnp.where(inside, 1.0, 0.0).astype(jnp.float32)


def kernel(output_roi):
    bbox = output_roi[:, 1:5].astype(jnp.int32)
    return pl.pallas_call(
        _body,
        grid=(N // B,),
        in_specs=[
            pl.BlockSpec((B, 4), lambda i: (i, 0), memory_space=pltpu.SMEM)
        ],
        out_specs=pl.BlockSpec((B, OUT_D, OUT_D), lambda i: (i, 0, 0)),
        out_shape=jax.ShapeDtypeStruct((N, OUT_D, OUT_D), jnp.float32),
    )(bbox)


# speedup vs baseline: 1.5291x; 1.1166x over previous
import jax
import jax.numpy as jnp
from jax.experimental import pallas as pl
from jax.experimental.pallas import tpu as pltpu

OUT_D = 128
N = 4096
B = 128


def _body(roi_ref, out_ref):
    r2 = jax.lax.broadcasted_iota(jnp.uint32, (OUT_D, OUT_D), 0)
    c2 = jax.lax.broadcasted_iota(jnp.uint32, (OUT_D, OUT_D), 1)
    for b in range(B):
        x = roi_ref[b, 0].astype(jnp.uint32)
        y = roi_ref[b, 1].astype(jnp.uint32)
        w = roi_ref[b, 2].astype(jnp.uint32)
        h = roi_ref[b, 3].astype(jnp.uint32)
        inside = ((r2 - y) <= h) & ((c2 - x) <= w)
        out_ref[b] = jnp.where(inside, 1.0, 0.0).astype(jnp.float32)


def kernel(output_roi):
    bbox = output_roi[:, 1:5].astype(jnp.int32)
    return pl.pallas_call(
        _body,
        grid=(N // B,),
        in_specs=[
            pl.BlockSpec((B, 4), lambda i: (i, 0), memory_space=pltpu.SMEM)
        ],
        out_specs=pl.BlockSpec((B, OUT_D, OUT_D), lambda i: (i, 0, 0)),
        out_shape=jax.ShapeDtypeStruct((N, OUT_D, OUT_D), jnp.float32),
    )(bbox)
